# Initial kernel scaffold; baseline (speedup 1.0000x reference)
import jax
import jax.numpy as jnp
from jax.experimental import pallas as pl


def kernel(index, scores):
    raise NotImplementedError("write your pallas kernel here")


# speedup vs baseline: 3.2057x; 3.2057x over previous
import jax
import jax.numpy as jnp
from jax.experimental import pallas as pl

B, H, L_Q, U, L_K = 4, 16, 4096, 128, 4096

ROWS_PER_BLOCK = 512


def _mask_kernel(idx_ref, out_ref):
    col = jax.lax.broadcasted_iota(jnp.int32, out_ref.shape, 1)
    out_ref[...] = col > idx_ref[...]


def kernel(index, scores):
    del scores
    n_rows = B * H * U
    idx = index.reshape(n_rows, 1).astype(jnp.int32)
    grid = (n_rows // ROWS_PER_BLOCK,)
    out = pl.pallas_call(
        _mask_kernel,
        grid=grid,
        in_specs=[pl.BlockSpec((ROWS_PER_BLOCK, 1), lambda i: (i, 0))],
        out_specs=pl.BlockSpec((ROWS_PER_BLOCK, L_K), lambda i: (i, 0)),
        out_shape=jax.ShapeDtypeStruct((n_rows, L_K), jnp.bool_),
    )(idx)
    return out.reshape(B, H, U, L_K)
